# Initial kernel scaffold; baseline (speedup 1.0000x reference)
import jax
import jax.numpy as jnp
from jax.experimental import pallas as pl


def kernel(latents, idx):
    raise NotImplementedError("write your pallas kernel here")


# speedup vs baseline: 2.4001x; 2.4001x over previous
import functools

import jax
import jax.numpy as jnp
from jax import lax
from jax.experimental import pallas as pl
from jax.experimental.pallas import tpu as pltpu
from jax.experimental.pallas import tpu_sc as plsc

NUM_ITEMS = 100000
D = 256
K = 100
B = 1024
NPAD = 100352
NT = 2048
BT = 8

_sc_info = plsc.get_sparse_core_info()
_NW = _sc_info.num_cores * _sc_info.num_subcores


def _make_sc_gather(nrows, chunk):
  assert nrows % (8 * _NW) == 0
  b_per_w = nrows // _NW
  assert b_per_w % chunk == 0
  nchunks = b_per_w // chunk
  mesh = plsc.VectorSubcoreMesh(core_axis_name="c", subcore_axis_name="s")

  @functools.partial(
      pl.kernel,
      mesh=mesh,
      out_type=jax.ShapeDtypeStruct((nrows, D), jnp.float32),
      scratch_types=[
          pltpu.VMEM((chunk,), jnp.int32),
          pltpu.VMEM((chunk, D), jnp.float32),
          pltpu.SemaphoreType.DMA,
      ],
  )
  def gather_kernel(table_hbm, idx_hbm, out_hbm, idx_v, rows_v, sem):
    wid = lax.axis_index("s") * _sc_info.num_cores + lax.axis_index("c")
    base = wid * b_per_w
    for c in range(nchunks):
      off = base + c * chunk
      pltpu.sync_copy(idx_hbm.at[pl.ds(off, chunk)], idx_v)
      pltpu.async_copy(table_hbm.at[idx_v], rows_v, sem).wait()
      pltpu.sync_copy(rows_v, out_hbm.at[pl.ds(off, chunk)])

  return gather_kernel


_gather_z = _make_sc_gather(B, 32)
_gather_nn = _make_sc_gather(B * K, 128)


def _dist_kernel(z_ref, lat_ref, out_ref):
  z = z_ref[...]
  latt = lat_ref[...]
  z2 = jnp.sum(z * z, axis=1, keepdims=True)
  l2 = jnp.sum(latt * latt, axis=1)[None, :]
  mm = lax.dot_general(z, latt, (((1,), (1,)), ((), ())),
                       preferred_element_type=jnp.float32)
  out_ref[...] = z2 + l2 - 2.0 * mm


def _distances(z, lat_pad):
  grid = NPAD // NT
  return pl.pallas_call(
      _dist_kernel,
      grid=(grid,),
      in_specs=[
          pl.BlockSpec((B, D), lambda j: (0, 0)),
          pl.BlockSpec((NT, D), lambda j: (j, 0)),
      ],
      out_specs=pl.BlockSpec((B, NT), lambda j: (0, j)),
      out_shape=jax.ShapeDtypeStruct((B, NPAD), jnp.float32),
  )(z, lat_pad)


_NSEL = K + 1


def _topk_kernel(d_ref, out_ref, scratch_ref):
  scratch_ref[...] = d_ref[...]
  col = lax.broadcasted_iota(jnp.int32, (BT, NPAD), 1)
  lane = lax.broadcasted_iota(jnp.int32, (BT, 128), 1)

  def body(k, acc):
    d = scratch_ref[...]
    m = jnp.min(d, axis=1, keepdims=True)
    cand = jnp.where(d <= m, col, NPAD)
    mi = jnp.min(cand, axis=1, keepdims=True)
    scratch_ref[...] = jnp.where(col == mi, jnp.inf, d)
    return jnp.where(lane == k, jnp.broadcast_to(mi, (BT, 128)), acc)

  out_ref[...] = lax.fori_loop(0, _NSEL, body, jnp.zeros((BT, 128), jnp.int32))


def _topk(d2):
  grid = B // BT
  return pl.pallas_call(
      _topk_kernel,
      grid=(grid,),
      in_specs=[pl.BlockSpec((BT, NPAD), lambda i: (i, 0))],
      out_specs=pl.BlockSpec((BT, 128), lambda i: (i, 0)),
      out_shape=jax.ShapeDtypeStruct((B, 128), jnp.int32),
      scratch_shapes=[pltpu.VMEM((BT, NPAD), jnp.float32)],
  )(d2)


def _tail_kernel(s_ref, z_ref, t_ref, f_ref, out_ref):
  z = z_ref[...]
  t = t_ref[...]
  f = f_ref[...]
  s_all = s_ref[...]
  smax = jnp.max(s_all, axis=1)
  smin = jnp.min(s_all, axis=1)
  diff = smax - smin
  zp = z + t * diff / 16.0 * f

  rowi = lax.broadcasted_iota(jnp.int32, (K, K + 1), 0)
  lanei = lax.broadcasted_iota(jnp.int32, (K, K + 1), 1)

  for i in range(BT):
    srow = s_all[i]
    g = lax.dot_general(srow, srow, (((1,), (1,)), ((), ())),
                        preferred_element_type=jnp.float32)
    zpi = zp[i][None, :]
    rhs = lax.dot_general(srow, zpi, (((1,), (1,)), ((), ())),
                          preferred_element_type=jnp.float32)
    a0 = jnp.concatenate([g, rhs], axis=1)

    def gj_step(j, a):
      rmask = (rowi == j).astype(jnp.float32)
      lmask = (lanei == j).astype(jnp.float32)
      r = jnp.sum(a * rmask, axis=0, keepdims=True)
      pivot = jnp.sum(r * lmask[0:1, :], axis=1, keepdims=True)
      rn = r / pivot
      c = jnp.sum(a * lmask, axis=1, keepdims=True)
      return a - c * rn + rmask * rn

    a = lax.fori_loop(0, K, gj_step, a0)
    w = jnp.sum(a * (lanei == K).astype(jnp.float32), axis=1, keepdims=True)
    outrow = lax.dot_general(w, srow, (((0,), (0,)), ((), ())),
                             preferred_element_type=jnp.float32)
    out_ref[i, :] = outrow[0]


def _tail(s, z, t, f):
  grid = B // BT
  return pl.pallas_call(
      _tail_kernel,
      grid=(grid,),
      in_specs=[
          pl.BlockSpec((BT, K, D), lambda i: (i, 0, 0)),
          pl.BlockSpec((BT, D), lambda i: (i, 0)),
          pl.BlockSpec((BT, D), lambda i: (i, 0)),
          pl.BlockSpec((BT, D), lambda i: (i, 0)),
      ],
      out_specs=pl.BlockSpec((BT, D), lambda i: (i, 0)),
      out_shape=jax.ShapeDtypeStruct((B, D), jnp.float32),
  )(s, z, t, f)


def kernel(latents, idx):
  idx = idx.astype(jnp.int32)
  lat_pad = jnp.pad(latents, ((0, NPAD - NUM_ITEMS), (0, 0)),
                    constant_values=1000.0)

  rk = jax.random.key(42)
  rk1, rk2 = jax.random.split(rk)
  flip = (jnp.where(jax.random.uniform(rk1, (B, D)) > 0.5, 1.0, 0.0) - 0.5) * 2.0
  tscale = 10.0 + jax.random.uniform(rk2, (B, D)) * 1.2

  z = _gather_z(latents, idx)
  d2 = _distances(z, lat_pad)
  sel = _topk(d2)
  nn_idx = sel[:, 1:_NSEL]
  s_flat = _gather_nn(latents, nn_idx.reshape(-1))
  s = s_flat.reshape(B, K, D)
  return _tail(s, z, tscale, flip)


# speedup vs baseline: 2.7091x; 1.1287x over previous
import functools

import jax
import jax.numpy as jnp
from jax import lax
from jax.experimental import pallas as pl
from jax.experimental.pallas import tpu as pltpu
from jax.experimental.pallas import tpu_sc as plsc

NUM_ITEMS = 100000
D = 256
K = 100
B = 1024
NPAD = 100352
NT = 2048
BT = 8

_sc_info = plsc.get_sparse_core_info()
_NW = _sc_info.num_cores * _sc_info.num_subcores


def _make_sc_gather(nrows, chunk):
  assert nrows % (8 * _NW) == 0
  b_per_w = nrows // _NW
  assert b_per_w % chunk == 0
  nchunks = b_per_w // chunk
  mesh = plsc.VectorSubcoreMesh(core_axis_name="c", subcore_axis_name="s")

  @functools.partial(
      pl.kernel,
      mesh=mesh,
      out_type=jax.ShapeDtypeStruct((nrows, D), jnp.float32),
      scratch_types=[
          pltpu.VMEM((chunk,), jnp.int32),
          pltpu.VMEM((chunk, D), jnp.float32),
          pltpu.SemaphoreType.DMA,
      ],
  )
  def gather_kernel(table_hbm, idx_hbm, out_hbm, idx_v, rows_v, sem):
    wid = lax.axis_index("s") * _sc_info.num_cores + lax.axis_index("c")
    base = wid * b_per_w
    for c in range(nchunks):
      off = base + c * chunk
      pltpu.sync_copy(idx_hbm.at[pl.ds(off, chunk)], idx_v)
      pltpu.async_copy(table_hbm.at[idx_v], rows_v, sem).wait()
      pltpu.sync_copy(rows_v, out_hbm.at[pl.ds(off, chunk)])

  return gather_kernel


_gather_z = _make_sc_gather(B, 32)
_gather_nn = _make_sc_gather(B * K, 128)


def _dist_kernel(z_ref, lat_ref, out_ref):
  z = z_ref[...]
  latt = lat_ref[...]
  z2 = jnp.sum(z * z, axis=1, keepdims=True)
  l2 = jnp.sum(latt * latt, axis=1)[None, :]
  mm = lax.dot_general(z, latt, (((1,), (1,)), ((), ())),
                       preferred_element_type=jnp.float32)
  out_ref[...] = z2 + l2 - 2.0 * mm


def _distances(z, lat_pad):
  grid = NPAD // NT
  return pl.pallas_call(
      _dist_kernel,
      grid=(grid,),
      in_specs=[
          pl.BlockSpec((B, D), lambda j: (0, 0)),
          pl.BlockSpec((NT, D), lambda j: (j, 0)),
      ],
      out_specs=pl.BlockSpec((B, NT), lambda j: (0, j)),
      out_shape=jax.ShapeDtypeStruct((B, NPAD), jnp.float32),
  )(z, lat_pad)


_NSEL = K + 1


_G = NPAD // 128


def _topk_kernel(d_ref, out_ref, s3_ref):
  d3 = d_ref[...].reshape(BT, _G, 128)
  s3_ref[...] = d3
  gm0 = jnp.min(d3, axis=2)
  giota = lax.broadcasted_iota(jnp.int32, (BT, _G), 1)
  lane = lax.broadcasted_iota(jnp.int32, (BT, 128), 1)
  lane1 = lax.broadcasted_iota(jnp.int32, (1, 128), 1)
  rowi = lax.broadcasted_iota(jnp.int32, (BT, 1), 0)

  def body(k, carry):
    gm, acc = carry
    m = jnp.min(gm, axis=1, keepdims=True)
    gi = jnp.min(jnp.where(gm <= m, giota, _G), axis=1, keepdims=True)
    li_all = jnp.zeros((BT, 1), jnp.int32)
    ngm_all = jnp.zeros((BT, 1), jnp.float32)
    for r in range(BT):
      gir = gi[r, 0]
      slab = s3_ref[r, pl.ds(gir, 1), :]
      mv = jnp.min(slab, axis=1, keepdims=True)
      li = jnp.min(jnp.where(slab <= mv, lane1, 128), axis=1, keepdims=True)
      ns = jnp.where(lane1 == li, jnp.inf, slab)
      s3_ref[r, pl.ds(gir, 1), :] = ns
      nm = jnp.min(ns, axis=1, keepdims=True)
      rsel = rowi == r
      li_all = jnp.where(rsel, jnp.broadcast_to(li, (BT, 1)), li_all)
      ngm_all = jnp.where(rsel, jnp.broadcast_to(nm, (BT, 1)), ngm_all)
    gm = jnp.where(giota == gi, ngm_all, gm)
    gidx = gi * 128 + li_all
    acc = jnp.where(lane == k, jnp.broadcast_to(gidx, (BT, 128)), acc)
    return gm, acc

  _, acc = lax.fori_loop(
      0, _NSEL, body, (gm0, jnp.zeros((BT, 128), jnp.int32)))
  out_ref[...] = acc


def _topk(d2):
  grid = B // BT
  return pl.pallas_call(
      _topk_kernel,
      grid=(grid,),
      in_specs=[pl.BlockSpec((BT, NPAD), lambda i: (i, 0))],
      out_specs=pl.BlockSpec((BT, 128), lambda i: (i, 0)),
      out_shape=jax.ShapeDtypeStruct((B, 128), jnp.int32),
      scratch_shapes=[pltpu.VMEM((BT, _G, 128), jnp.float32)],
  )(d2)


def _tail_kernel(s_ref, z_ref, t_ref, f_ref, out_ref):
  z = z_ref[...]
  t = t_ref[...]
  f = f_ref[...]
  s_all = s_ref[...]
  smax = jnp.max(s_all, axis=1)
  smin = jnp.min(s_all, axis=1)
  diff = smax - smin
  zp = z + t * diff / 16.0 * f

  rowi = lax.broadcasted_iota(jnp.int32, (1, K, K + 1), 1)
  lanei = lax.broadcasted_iota(jnp.int32, (1, K, K + 1), 2)

  augs = []
  for i in range(BT):
    srow = s_all[i]
    g = lax.dot_general(srow, srow, (((1,), (1,)), ((), ())),
                        preferred_element_type=jnp.float32)
    zpi = zp[i][None, :]
    rhs = lax.dot_general(srow, zpi, (((1,), (1,)), ((), ())),
                          preferred_element_type=jnp.float32)
    augs.append(jnp.concatenate([g, rhs], axis=1)[None])
  a0 = jnp.concatenate(augs, axis=0)

  def gj_step(j, a):
    rmask = (rowi == j).astype(jnp.float32)
    lmask = (lanei == j).astype(jnp.float32)
    r = jnp.sum(a * rmask, axis=1, keepdims=True)
    pivot = jnp.sum(r * lmask[:, 0:1, :], axis=2, keepdims=True)
    rn = r / pivot
    c = jnp.sum(a * lmask, axis=2, keepdims=True)
    return a - c * rn + rmask * rn

  a = lax.fori_loop(0, K, gj_step, a0)
  w = jnp.sum(a * (lanei == K).astype(jnp.float32), axis=2, keepdims=True)
  for i in range(BT):
    outrow = lax.dot_general(w[i], s_all[i], (((0,), (0,)), ((), ())),
                             preferred_element_type=jnp.float32)
    out_ref[i, :] = outrow[0]


def _tail(s, z, t, f):
  grid = B // BT
  return pl.pallas_call(
      _tail_kernel,
      grid=(grid,),
      in_specs=[
          pl.BlockSpec((BT, K, D), lambda i: (i, 0, 0)),
          pl.BlockSpec((BT, D), lambda i: (i, 0)),
          pl.BlockSpec((BT, D), lambda i: (i, 0)),
          pl.BlockSpec((BT, D), lambda i: (i, 0)),
      ],
      out_specs=pl.BlockSpec((BT, D), lambda i: (i, 0)),
      out_shape=jax.ShapeDtypeStruct((B, D), jnp.float32),
  )(s, z, t, f)


def kernel(latents, idx):
  idx = idx.astype(jnp.int32)
  lat_pad = jnp.pad(latents, ((0, NPAD - NUM_ITEMS), (0, 0)),
                    constant_values=1000.0)

  rk = jax.random.key(42)
  rk1, rk2 = jax.random.split(rk)
  flip = (jnp.where(jax.random.uniform(rk1, (B, D)) > 0.5, 1.0, 0.0) - 0.5) * 2.0
  tscale = 10.0 + jax.random.uniform(rk2, (B, D)) * 1.2

  z = _gather_z(latents, idx)
  d2 = _distances(z, lat_pad)
  sel = _topk(d2)
  nn_idx = sel[:, 1:_NSEL]
  s_flat = _gather_nn(latents, nn_idx.reshape(-1))
  s = s_flat.reshape(B, K, D)
  return _tail(s, z, tscale, flip)
